# Initial kernel scaffold; baseline (speedup 1.0000x reference)
import functools

import jax
import jax.numpy as jnp
from jax import lax
from jax.experimental import pallas as pl
from jax.experimental.pallas import tpu as pltpu
from jax.experimental.pallas import tpu_sc as plsc

N_FRAMES = 32
B, S, D = 16384, 200, 4
N_TOTAL = B * S
NC, NS = 2, 16
NW = NC * NS
PER_W = N_TOTAL // NW
CH = 10240
NCHUNK = PER_W // CH
GROUPS = CH // 16

_mesh = plsc.VectorSubcoreMesh(
    core_axis_name="c", subcore_axis_name="s", num_cores=NC, num_subcores=NS
)


@functools.partial(
    pl.kernel,
    mesh=_mesh,
    out_type=jax.ShapeDtypeStruct((N_TOTAL * D,), jnp.float32),
    scratch_types=[
        pltpu.VMEM((N_FRAMES,), jnp.int32),
        pltpu.VMEM((N_FRAMES * D,), jnp.float32),
        pltpu.VMEM((N_FRAMES * D,), jnp.float32),
        pltpu.VMEM((CH,), jnp.int32),
        pltpu.VMEM((CH,), jnp.int32),
        pltpu.VMEM((CH * D,), jnp.float32),
        pltpu.VMEM((CH * D,), jnp.float32),
        pltpu.SemaphoreType.DMA,
        pltpu.SemaphoreType.DMA,
        pltpu.SemaphoreType.DMA,
        pltpu.SemaphoreType.DMA,
    ],
)
def _sc_gather(idx_hbm, fmi_hbm, intr_hbm, out_hbm,
               fmi_v, intr_v, tbl, ib0, ib1, ob0, ob1,
               si0, si1, so0, so1):
    wid = lax.axis_index("s") * NC + lax.axis_index("c")
    base = wid * PER_W

    pltpu.sync_copy(fmi_hbm, fmi_v)
    pltpu.sync_copy(intr_hbm, intr_v)
    lanes = lax.iota(jnp.int32, 16)
    for h in range(N_FRAMES // 16):
        f4 = fmi_v[pl.ds(h * 16, 16)] << 2
        for c in range(D):
            vals = plsc.load_gather(intr_v, [f4 + c])
            plsc.store_scatter(tbl, [lanes * D + (h * 16 * D + c)], vals)

    ibufs = [ib0, ib1]
    obufs = [ob0, ob1]
    isems = [si0, si1]
    osems = [so0, so1]
    in_d = [None, None]
    out_d = [None, None]

    st = [lanes * D + c for c in range(D)]

    in_d[0] = pltpu.async_copy(idx_hbm.at[pl.ds(base, CH)], ibufs[0], isems[0])
    for k in range(NCHUNK):
        b = k % 2
        if k + 1 < NCHUNK:
            in_d[1 - b] = pltpu.async_copy(
                idx_hbm.at[pl.ds(base + (k + 1) * CH, CH)], ibufs[1 - b],
                isems[1 - b])
        in_d[b].wait()
        if k >= 2:
            out_d[b].wait()
        ib = ibufs[b]
        ob = obufs[b]

        @pl.loop(0, GROUPS, unroll=4)
        def _group(g):
            f4 = ib[pl.ds(g * 16, 16)] << 2
            o = g * (16 * D)
            for c in range(D):
                vals = plsc.load_gather(tbl, [f4 + c])
                plsc.store_scatter(ob, [st[c] + o], vals)

        out_d[b] = pltpu.async_copy(
            ob, out_hbm.at[pl.ds((base + k * CH) * D, CH * D)], osems[b])
    out_d[(NCHUNK - 2) % 2].wait()
    out_d[(NCHUNK - 1) % 2].wait()


def kernel(frame_id, frame_mapping_inv, intrinsics):
    out_flat = _sc_gather(
        frame_id.reshape(-1),
        frame_mapping_inv,
        intrinsics.reshape(-1),
    )
    return out_flat.reshape(B, S, D)


# speedup vs baseline: 5.1978x; 5.1978x over previous
import functools

import jax
import jax.numpy as jnp
from jax import lax
from jax.experimental import pallas as pl
from jax.experimental.pallas import tpu as pltpu
from jax.experimental.pallas import tpu_sc as plsc

N_FRAMES = 32
B, S, D = 16384, 200, 4
N_TOTAL = B * S
NC, NS = 2, 16
NW = NC * NS
PER_W = N_TOTAL // NW
CH = 10240
NCHUNK = PER_W // CH
GROUPS = CH // 16

_mesh = plsc.VectorSubcoreMesh(
    core_axis_name="c", subcore_axis_name="s", num_cores=NC, num_subcores=NS
)


@functools.partial(
    pl.kernel,
    mesh=_mesh,
    compiler_params=pltpu.CompilerParams(needs_layout_passes=False),
    out_type=jax.ShapeDtypeStruct((N_TOTAL * D,), jnp.float32),
    scratch_types=[
        pltpu.VMEM((N_FRAMES,), jnp.int32),
        pltpu.VMEM((N_FRAMES * D,), jnp.float32),
        pltpu.VMEM((N_FRAMES * D,), jnp.float32),
        pltpu.VMEM((CH,), jnp.int32),
        pltpu.VMEM((CH,), jnp.int32),
        pltpu.VMEM((CH * D,), jnp.float32),
        pltpu.VMEM((CH * D,), jnp.float32),
        pltpu.SemaphoreType.DMA,
        pltpu.SemaphoreType.DMA,
        pltpu.SemaphoreType.DMA,
        pltpu.SemaphoreType.DMA,
    ],
)
def _sc_gather(idx_hbm, fmi_hbm, intr_hbm, out_hbm,
               fmi_v, intr_v, tbl, ib0, ib1, ob0, ob1,
               si0, si1, so0, so1):
    wid = lax.axis_index("s") * NC + lax.axis_index("c")
    base = wid * PER_W

    pltpu.sync_copy(fmi_hbm, fmi_v)
    pltpu.sync_copy(intr_hbm, intr_v)
    lanes = lax.iota(jnp.int32, 16)
    for h in range(N_FRAMES // 16):
        f4 = fmi_v[pl.ds(h * 16, 16)] << 2
        for c in range(D):
            vals = plsc.load_gather(intr_v, [f4 + c])
            plsc.store_scatter(tbl, [lanes * D + (h * 16 * D + c)], vals)

    ibufs = [ib0, ib1]
    obufs = [ob0, ob1]
    isems = [si0, si1]
    osems = [so0, so1]
    in_d = [None, None]
    out_d = [None, None]

    st = [lanes * D + c for c in range(D)]

    in_d[0] = pltpu.async_copy(idx_hbm.at[pl.ds(base, CH)], ibufs[0], isems[0])
    for k in range(NCHUNK):
        b = k % 2
        if k + 1 < NCHUNK:
            in_d[1 - b] = pltpu.async_copy(
                idx_hbm.at[pl.ds(base + (k + 1) * CH, CH)], ibufs[1 - b],
                isems[1 - b])
        in_d[b].wait()
        if k >= 2:
            out_d[b].wait()
        ib = ibufs[b]
        ob = obufs[b]

        @pl.loop(0, GROUPS, unroll=4)
        def _group(g):
            f4 = ib[pl.ds(g * 16, 16)] << 2
            o = g * (16 * D)
            for c in range(D):
                vals = plsc.load_gather(tbl, [f4 + c])
                plsc.store_scatter(ob, [st[c] + o], vals)

        out_d[b] = pltpu.async_copy(
            ob, out_hbm.at[pl.ds((base + k * CH) * D, CH * D)], osems[b])
    out_d[(NCHUNK - 2) % 2].wait()
    out_d[(NCHUNK - 1) % 2].wait()


def kernel(frame_id, frame_mapping_inv, intrinsics):
    out_flat = _sc_gather(
        frame_id.reshape(-1),
        frame_mapping_inv,
        intrinsics.reshape(-1),
    )
    return out_flat.reshape(B, S, D)


# speedup vs baseline: 81.0532x; 15.5939x over previous
import functools

import jax
import jax.numpy as jnp
from jax import lax
from jax.experimental import pallas as pl
from jax.experimental.pallas import tpu as pltpu
from jax.experimental.pallas import tpu_sc as plsc

N_FRAMES = 32
B, S, D = 16384, 200, 4
N_TOTAL = B * S
NC, NS = 2, 16
NW = NC * NS
PER_W = N_TOTAL // NW
CH = 10240
NCHUNK = PER_W // CH
BLOCKS = CH // 128

_mesh = plsc.VectorSubcoreMesh(
    core_axis_name="c", subcore_axis_name="s", num_cores=NC, num_subcores=NS
)


@functools.partial(
    pl.kernel,
    mesh=_mesh,
    compiler_params=pltpu.CompilerParams(needs_layout_passes=False),
    out_type=jax.ShapeDtypeStruct((N_TOTAL * D,), jnp.float32),
    scratch_types=[
        pltpu.VMEM((N_FRAMES,), jnp.int32),
        pltpu.VMEM((N_FRAMES * D,), jnp.float32),
        [pltpu.VMEM((N_FRAMES,), jnp.float32) for _ in range(D)],
        pltpu.VMEM((CH,), jnp.int32),
        pltpu.VMEM((CH,), jnp.int32),
        pltpu.VMEM((CH * D,), jnp.float32),
        pltpu.VMEM((CH * D,), jnp.float32),
        pltpu.SemaphoreType.DMA,
        pltpu.SemaphoreType.DMA,
        pltpu.SemaphoreType.DMA,
        pltpu.SemaphoreType.DMA,
    ],
)
def _sc_gather(idx_hbm, fmi_hbm, intr_hbm, out_hbm,
               fmi_v, intr_v, tbls, ib0, ib1, ob0, ob1,
               si0, si1, so0, so1):
    wid = lax.axis_index("s") * NC + lax.axis_index("c")
    base = wid * PER_W

    pltpu.sync_copy(fmi_hbm, fmi_v)
    pltpu.sync_copy(intr_hbm, intr_v)
    for h in range(N_FRAMES // 16):
        f4 = fmi_v[pl.ds(h * 16, 16)] << 2
        for c in range(D):
            tbls[c][pl.ds(h * 16, 16)] = plsc.load_gather(intr_v, [f4 + c])

    ibufs = [ib0, ib1]
    obufs = [ob0, ob1]
    isems = [si0, si1]
    osems = [so0, so1]
    in_d = [None, None]
    out_d = [None, None]

    in_d[0] = pltpu.async_copy(idx_hbm.at[pl.ds(base, CH)], ibufs[0], isems[0])
    for k in range(NCHUNK):
        b = k % 2
        if k + 1 < NCHUNK:
            in_d[1 - b] = pltpu.async_copy(
                idx_hbm.at[pl.ds(base + (k + 1) * CH, CH)], ibufs[1 - b],
                isems[1 - b])
        in_d[b].wait()
        if k >= 2:
            out_d[b].wait()
        ib = ibufs[b]
        ob = obufs[b]

        @pl.loop(0, BLOCKS)
        def _block(blk):
            ioff = blk * 128
            ooff = blk * (128 * D)
            for a in range(8):
                idx = ib[pl.ds(ioff + a * 16, 16)]
                for c in range(D):
                    ob[pl.ds(ooff + c * 128 + a * 16, 16)] = (
                        plsc.load_gather(tbls[c], [idx]))

        out_d[b] = pltpu.async_copy(
            ob, out_hbm.at[pl.ds((base + k * CH) * D, CH * D)], osems[b])
    out_d[(NCHUNK - 2) % 2].wait()
    out_d[(NCHUNK - 1) % 2].wait()


def kernel(frame_id, frame_mapping_inv, intrinsics):
    idx_flat = frame_id.T.reshape(-1)
    o = _sc_gather(idx_flat, frame_mapping_inv, intrinsics.reshape(-1))
    o = o.reshape(S, B // 128, D, 128)
    return o.transpose(1, 3, 0, 2).reshape(B, S, D)


# speedup vs baseline: 186.0869x; 2.2959x over previous
import functools

import jax
import jax.numpy as jnp
from jax import lax
from jax.experimental import pallas as pl
from jax.experimental.pallas import tpu as pltpu
from jax.experimental.pallas import tpu_sc as plsc

N_FRAMES = 32
B, S, D = 16384, 200, 4
N_TOTAL = B * S
NC, NS = 2, 16
NW = NC * NS
PER_W = N_TOTAL // NW
CH = 10240
NCHUNK = PER_W // CH
BLOCKS = CH // 128

_mesh = plsc.VectorSubcoreMesh(
    core_axis_name="c", subcore_axis_name="s", num_cores=NC, num_subcores=NS
)


@functools.partial(
    pl.kernel,
    mesh=_mesh,
    compiler_params=pltpu.CompilerParams(needs_layout_passes=False),
    out_type=jax.ShapeDtypeStruct((N_TOTAL * D,), jnp.float32),
    scratch_types=[
        pltpu.VMEM((N_FRAMES,), jnp.int32),
        pltpu.VMEM((N_FRAMES * D,), jnp.float32),
        [pltpu.VMEM((N_FRAMES,), jnp.float32) for _ in range(D)],
        pltpu.VMEM((CH,), jnp.int32),
        pltpu.VMEM((CH,), jnp.int32),
        pltpu.VMEM((CH * D,), jnp.float32),
        pltpu.VMEM((CH * D,), jnp.float32),
        pltpu.SemaphoreType.DMA,
        pltpu.SemaphoreType.DMA,
        pltpu.SemaphoreType.DMA,
        pltpu.SemaphoreType.DMA,
    ],
)
def _sc_gather(idx_hbm, fmi_hbm, intr_hbm, out_hbm,
               fmi_v, intr_v, tbls, ib0, ib1, ob0, ob1,
               si0, si1, so0, so1):
    wid = lax.axis_index("s") * NC + lax.axis_index("c")
    base = wid * PER_W

    pltpu.sync_copy(fmi_hbm, fmi_v)
    pltpu.sync_copy(intr_hbm, intr_v)
    for h in range(N_FRAMES // 16):
        f4 = fmi_v[pl.ds(h * 16, 16)] << 2
        for c in range(D):
            tbls[c][pl.ds(h * 16, 16)] = plsc.load_gather(intr_v, [f4 + c])

    ibufs = [ib0, ib1]
    obufs = [ob0, ob1]
    isems = [si0, si1]
    osems = [so0, so1]
    in_d = [None, None]
    out_d = [None, None]

    in_d[0] = pltpu.async_copy(idx_hbm.at[pl.ds(base, CH)], ibufs[0], isems[0])
    for k in range(NCHUNK):
        b = k % 2
        if k + 1 < NCHUNK:
            in_d[1 - b] = pltpu.async_copy(
                idx_hbm.at[pl.ds(base + (k + 1) * CH, CH)], ibufs[1 - b],
                isems[1 - b])
        in_d[b].wait()
        if k >= 2:
            out_d[b].wait()
        ib = ibufs[b]
        ob = obufs[b]

        @plsc.parallel_loop(0, BLOCKS, unroll=4)
        def _block(blk):
            ioff = blk * 128
            ooff = blk * (128 * D)
            idxs = [ib[pl.ds(ioff + a * 16, 16)] for a in range(8)]
            vals = [[plsc.load_gather(tbls[c], [idxs[a]]) for c in range(D)]
                    for a in range(8)]
            for a in range(8):
                for c in range(D):
                    ob[pl.ds(ooff + c * 128 + a * 16, 16)] = vals[a][c]

        out_d[b] = pltpu.async_copy(
            ob, out_hbm.at[pl.ds((base + k * CH) * D, CH * D)], osems[b])
    out_d[(NCHUNK - 2) % 2].wait()
    out_d[(NCHUNK - 1) % 2].wait()


def kernel(frame_id, frame_mapping_inv, intrinsics):
    idx_flat = frame_id.T.reshape(-1)
    o = _sc_gather(idx_flat, frame_mapping_inv, intrinsics.reshape(-1))
    o = o.reshape(S, B // 128, D, 128)
    return o.transpose(1, 3, 0, 2).reshape(B, S, D)
